# baseline (speedup 1.0000x reference)
import functools

import jax
import jax.numpy as jnp
from jax import lax
from jax.experimental import pallas as pl
from jax.experimental.pallas import tpu as pltpu
from jax.experimental.pallas import tpu_sc as plsc

_N = 5000
_N_PAD = 5120
_BLK = 1280
_NBLK = _N_PAD // _BLK
_IOU_T = 0.3
_SCORE_T = 0.05

_NC = 2
_NS = 16
_L = 16
_NW = _NC * _NS
_RPW = _N_PAD // _NW
_CHUNK = 80


def _gather_body(btfin_hbm, scores_hbm, order_hbm, btf_hbm, sp_hbm,
                 ord_v, idx_v, btl_v, spl_v, sem):
    wid = lax.axis_index("s") * _NC + lax.axis_index("c")
    base = wid * _RPW
    pltpu.sync_copy(order_hbm.at[pl.ds(base, _RPW)], ord_v)
    for c in range(4):
        for g in range(_RPW // _L):
            idx_v[pl.ds(c * _RPW + g * _L, _L)] = (
                ord_v[pl.ds(g * _L, _L)] + c * _N)
    copies = []
    for g in range(_RPW // _CHUNK):
        copies.append(pltpu.async_copy(
            scores_hbm.at[ord_v.at[pl.ds(g * _CHUNK, _CHUNK)]],
            spl_v.at[pl.ds(g * _CHUNK, _CHUNK)], sem))
    for c in range(4):
        for g in range(_RPW // _CHUNK):
            copies.append(pltpu.async_copy(
                btfin_hbm.at[idx_v.at[pl.ds(c * _RPW + g * _CHUNK, _CHUNK)]],
                btl_v.at[pl.ds(c * _RPW + g * _CHUNK, _CHUNK)], sem))
    for cp in copies:
        cp.wait()
    outs = [pltpu.async_copy(btl_v.at[pl.ds(c * _RPW, _RPW)],
                             btf_hbm.at[pl.ds(c * _N_PAD + base, _RPW)], sem)
            for c in range(4)]
    outs.append(pltpu.async_copy(spl_v, sp_hbm.at[pl.ds(base, _RPW)], sem))
    for cp in outs:
        cp.wait()


_gather_layout = functools.partial(
    pl.kernel,
    mesh=plsc.VectorSubcoreMesh(core_axis_name="c", subcore_axis_name="s"),
    out_type=[
        jax.ShapeDtypeStruct((4 * _N_PAD,), jnp.float32),
        jax.ShapeDtypeStruct((_N_PAD,), jnp.float32),
    ],
    scratch_types=[
        pltpu.VMEM((_RPW,), jnp.int32),
        pltpu.VMEM((4 * _RPW,), jnp.int32),
        pltpu.VMEM((4 * _RPW,), jnp.float32),
        pltpu.VMEM((_RPW,), jnp.float32),
        pltpu.SemaphoreType.DMA,
    ],
)(_gather_body)


def _iou_tile_mask(cy1, cx1, cy2, cx2, carea, ry1, rx1, ry2, rx2, rarea):
    tly = jnp.maximum(cy1, ry1)
    tlx = jnp.maximum(cx1, rx1)
    bry = jnp.minimum(cy2, ry2)
    brx = jnp.minimum(cx2, rx2)
    h = jnp.maximum(bry - tly, 0.0)
    w = jnp.maximum(brx - tlx, 0.0)
    inter = h * w
    iou = inter / (carea + rarea - inter + 1e-9)
    return (iou > _IOU_T).astype(jnp.float32)


def _nms_body(bt_ref, s_ref, keep_ref, bm_ref, sm_ref):
    keep_ref[...] = (s_ref[...] > _SCORE_T).astype(jnp.float32)

    row_id = jax.lax.broadcasted_iota(jnp.int32, (_BLK, _BLK), 0)
    col_id = jax.lax.broadcasted_iota(jnp.int32, (_BLK, _BLK), 1)
    tri = (col_id < row_id).astype(jnp.float32)
    eyeb = (row_id == col_id).astype(jnp.float32)

    def coord_rows(jb):
        rb = bt_ref[:, pl.ds(jb, _BLK)]
        return rb[0:1, :], rb[1:2, :], rb[2:3, :], rb[3:4, :]

    def t_col(rowvec):
        return jax.lax.dot_general(eyeb, rowvec, (((1,), (1,)), ((), ())),
                                   precision=jax.lax.Precision.HIGHEST,
                                   preferred_element_type=jnp.float32)

    for k in range(_NBLK):
        base = k * _BLK
        by1, bx1, by2, bx2 = coord_rows(base)
        cy1, cx1, cy2, cx2 = t_col(by1), t_col(bx1), t_col(by2), t_col(bx2)
        carea = (cy2 - cy1) * (cx2 - cx1)
        k0 = keep_ref[pl.ds(base, _BLK), :]

        def cross_body(j, acc, cy1=cy1, cx1=cx1, cy2=cy2, cx2=cx2,
                       carea=carea):
            ry1, rx1, ry2, rx2 = coord_rows(j * _BLK)
            rarea = (ry2 - ry1) * (rx2 - rx1)
            m = _iou_tile_mask(cy1, cx1, cy2, cx2, carea,
                               ry1, rx1, ry2, rx2, rarea)
            kprev = keep_ref[pl.ds(j * _BLK, _BLK), :]
            return acc + jax.lax.dot(m, kprev,
                                     preferred_element_type=jnp.float32)

        sup = jax.lax.fori_loop(0, k, cross_body,
                                jnp.zeros((_BLK, 1), jnp.float32))
        k0_eff = jnp.where(sup > 0.5, 0.0, k0)

        rarea = (by2 - by1) * (bx2 - bx1)
        m_self = _iou_tile_mask(cy1, cx1, cy2, cx2, carea,
                                by1, bx1, by2, bx2, rarea) * tri

        def fp_cond(st):
            t, _, changed = st
            return changed & (t < _BLK + 2)

        def fp_body(st, m_self=m_self, k0_eff=k0_eff):
            t, kc, _ = st
            sup2 = jax.lax.dot(m_self, kc, preferred_element_type=jnp.float32)
            kn = jnp.where(sup2 > 0.5, 0.0, k0_eff)
            return (t + 1, kn, jnp.any(kn != kc))

        _, kfin, _ = jax.lax.while_loop(
            fp_cond, fp_body, (0, k0_eff, jnp.bool_(True)))
        keep_ref[pl.ds(base, _BLK), :] = kfin

        bblk = bt_ref[:, pl.ds(base, _BLK)]
        brows = jax.lax.dot_general(eyeb, bblk, (((1,), (1,)), ((), ())),
                                    precision=jax.lax.Precision.HIGHEST,
                                    preferred_element_type=jnp.float32)
        bm_ref[pl.ds(base, _BLK), :] = brows * kfin

    sm_ref[...] = s_ref[...] * keep_ref[...]


def _run_nms(btp, sp, interpret=False):
    return pl.pallas_call(
        _nms_body,
        out_shape=[
            jax.ShapeDtypeStruct((_N_PAD, 1), jnp.float32),
            jax.ShapeDtypeStruct((_N_PAD, 4), jnp.float32),
            jax.ShapeDtypeStruct((_N_PAD, 1), jnp.float32),
        ],
        interpret=interpret,
    )(btp, sp)


def kernel(boxes, scores):
    order = jnp.argsort(-scores)
    orderp = jnp.pad(order, ((0, _N_PAD - _N),))
    btfin = boxes.T.reshape(4 * _N)
    btf, spf = _gather_layout(btfin, scores, orderp)
    _, bm, sm = _run_nms(btf.reshape(4, _N_PAD), spf[:, None])
    return jnp.concatenate([bm[:_N], sm[:_N]], axis=1)


# speedup vs baseline: 1.1019x; 1.1019x over previous
import functools

import jax
import jax.numpy as jnp
from jax import lax
from jax.experimental import pallas as pl
from jax.experimental.pallas import tpu as pltpu
from jax.experimental.pallas import tpu_sc as plsc

_N = 5000
_N_PAD = 5120
_BLK = 1024
_NBLK = _N_PAD // _BLK
_IOU_T = 0.3
_SCORE_T = 0.05

_NC = 2
_NS = 16
_L = 16
_NW = _NC * _NS
_RPW = _N_PAD // _NW
_CHUNK = 80


def _gather_body(btfin_hbm, scores_hbm, order_hbm, btf_hbm, sp_hbm,
                 ord_v, idx_v, btl_v, spl_v, sem):
    wid = lax.axis_index("s") * _NC + lax.axis_index("c")
    base = wid * _RPW
    pltpu.sync_copy(order_hbm.at[pl.ds(base, _RPW)], ord_v)
    for c in range(4):
        for g in range(_RPW // _L):
            idx_v[pl.ds(c * _RPW + g * _L, _L)] = (
                ord_v[pl.ds(g * _L, _L)] + c * _N)
    copies = []
    for g in range(_RPW // _CHUNK):
        copies.append(pltpu.async_copy(
            scores_hbm.at[ord_v.at[pl.ds(g * _CHUNK, _CHUNK)]],
            spl_v.at[pl.ds(g * _CHUNK, _CHUNK)], sem))
    for c in range(4):
        for g in range(_RPW // _CHUNK):
            copies.append(pltpu.async_copy(
                btfin_hbm.at[idx_v.at[pl.ds(c * _RPW + g * _CHUNK, _CHUNK)]],
                btl_v.at[pl.ds(c * _RPW + g * _CHUNK, _CHUNK)], sem))
    for cp in copies:
        cp.wait()
    outs = [pltpu.async_copy(btl_v.at[pl.ds(c * _RPW, _RPW)],
                             btf_hbm.at[pl.ds(c * _N_PAD + base, _RPW)], sem)
            for c in range(4)]
    outs.append(pltpu.async_copy(spl_v, sp_hbm.at[pl.ds(base, _RPW)], sem))
    for cp in outs:
        cp.wait()


_gather_layout = functools.partial(
    pl.kernel,
    mesh=plsc.VectorSubcoreMesh(core_axis_name="c", subcore_axis_name="s"),
    out_type=[
        jax.ShapeDtypeStruct((4 * _N_PAD,), jnp.float32),
        jax.ShapeDtypeStruct((_N_PAD,), jnp.float32),
    ],
    scratch_types=[
        pltpu.VMEM((_RPW,), jnp.int32),
        pltpu.VMEM((4 * _RPW,), jnp.int32),
        pltpu.VMEM((4 * _RPW,), jnp.float32),
        pltpu.VMEM((_RPW,), jnp.float32),
        pltpu.SemaphoreType.DMA,
    ],
)(_gather_body)


def _iou_tile_mask(cy1, cx1, cy2, cx2, carea, ry1, rx1, ry2, rx2, rarea):
    tly = jnp.maximum(cy1, ry1)
    tlx = jnp.maximum(cx1, rx1)
    bry = jnp.minimum(cy2, ry2)
    brx = jnp.minimum(cx2, rx2)
    h = jnp.maximum(bry - tly, 0.0)
    w = jnp.maximum(brx - tlx, 0.0)
    inter = h * w
    iou = inter / (carea + rarea - inter + 1e-9)
    return (iou > _IOU_T).astype(jnp.float32)


def _nms_body(bt_ref, s_ref, out_ref, keep_ref):
    keep_ref[...] = (s_ref[...] > _SCORE_T).astype(jnp.float32)

    row_id = jax.lax.broadcasted_iota(jnp.int32, (_BLK, _BLK), 0)
    col_id = jax.lax.broadcasted_iota(jnp.int32, (_BLK, _BLK), 1)
    tri = (col_id < row_id).astype(jnp.float32)
    eyeb = (row_id == col_id).astype(jnp.float32)

    def coord_rows(jb):
        rb = bt_ref[:, pl.ds(jb, _BLK)]
        return rb[0:1, :], rb[1:2, :], rb[2:3, :], rb[3:4, :]

    def t_col(rowvec):
        return jax.lax.dot_general(eyeb, rowvec, (((1,), (1,)), ((), ())),
                                   precision=jax.lax.Precision.HIGHEST,
                                   preferred_element_type=jnp.float32)

    for k in range(_NBLK):
        base = k * _BLK
        by1, bx1, by2, bx2 = coord_rows(base)
        cy1, cx1, cy2, cx2 = t_col(by1), t_col(bx1), t_col(by2), t_col(bx2)
        carea = (cy2 - cy1) * (cx2 - cx1)
        k0 = keep_ref[pl.ds(base, _BLK), :]

        def cross_body(j, acc, cy1=cy1, cx1=cx1, cy2=cy2, cx2=cx2,
                       carea=carea):
            ry1, rx1, ry2, rx2 = coord_rows(j * _BLK)
            rarea = (ry2 - ry1) * (rx2 - rx1)
            m = _iou_tile_mask(cy1, cx1, cy2, cx2, carea,
                               ry1, rx1, ry2, rx2, rarea)
            kprev = keep_ref[pl.ds(j * _BLK, _BLK), :]
            return acc + jax.lax.dot(m, kprev,
                                     preferred_element_type=jnp.float32)

        sup = jax.lax.fori_loop(0, k, cross_body,
                                jnp.zeros((_BLK, 1), jnp.float32))
        k0_eff = jnp.where(sup > 0.5, 0.0, k0)

        rarea = (by2 - by1) * (bx2 - bx1)
        m_self = _iou_tile_mask(cy1, cx1, cy2, cx2, carea,
                                by1, bx1, by2, bx2, rarea) * tri

        def fp_cond(st):
            t, _, changed = st
            return changed & (t < _BLK + 2)

        def fp_body(st, m_self=m_self, k0_eff=k0_eff):
            t, kc, _ = st
            sup2 = jax.lax.dot(m_self, kc, preferred_element_type=jnp.float32)
            kn = jnp.where(sup2 > 0.5, 0.0, k0_eff)
            return (t + 1, kn, jnp.any(kn != kc))

        _, kfin, _ = jax.lax.while_loop(
            fp_cond, fp_body, (0, k0_eff, jnp.bool_(True)))
        keep_ref[pl.ds(base, _BLK), :] = kfin

        bblk = bt_ref[:, pl.ds(base, _BLK)]
        brows = jax.lax.dot_general(eyeb, bblk, (((1,), (1,)), ((), ())),
                                    precision=jax.lax.Precision.HIGHEST,
                                    preferred_element_type=jnp.float32)
        srows = s_ref[pl.ds(base, _BLK), :]
        out_ref[pl.ds(base, _BLK), :] = (
            jnp.concatenate([brows, srows], axis=1) * kfin)


def _run_nms(btp, sp, interpret=False):
    return pl.pallas_call(
        _nms_body,
        out_shape=jax.ShapeDtypeStruct((_N_PAD, 5), jnp.float32),
        scratch_shapes=[pltpu.VMEM((_N_PAD, 1), jnp.float32)],
        interpret=interpret,
    )(btp, sp)


def kernel(boxes, scores):
    order = jnp.argsort(-scores)
    orderp = jnp.pad(order, ((0, _N_PAD - _N),))
    btfin = boxes.T.reshape(4 * _N)
    btf, spf = _gather_layout(btfin, scores, orderp)
    out = _run_nms(btf.reshape(4, _N_PAD), spf[:, None])
    return out[:_N]


# speedup vs baseline: 1.1092x; 1.0067x over previous
import functools

import jax
import jax.numpy as jnp
from jax import lax
from jax.experimental import pallas as pl
from jax.experimental.pallas import tpu as pltpu
from jax.experimental.pallas import tpu_sc as plsc

_N = 5000
_N_PAD = 5120
_BLK = 1024
_NBLK = _N_PAD // _BLK
_IOU_T = 0.3
_SCORE_T = 0.05

_NC = 2
_NS = 16
_L = 16
_NW = _NC * _NS
_RPW = _N_PAD // _NW
_CHUNK = 80


def _gather_body(btfin_hbm, scores_hbm, order_hbm, btf_hbm, sp_hbm,
                 ord_v, idx_v, btl_v, spl_v, sem):
    wid = lax.axis_index("s") * _NC + lax.axis_index("c")
    base = wid * _RPW
    pltpu.sync_copy(order_hbm.at[pl.ds(base, _RPW)], ord_v)
    copies = []
    for g in range(_RPW // _CHUNK):
        copies.append(pltpu.async_copy(
            scores_hbm.at[ord_v.at[pl.ds(g * _CHUNK, _CHUNK)]],
            spl_v.at[pl.ds(g * _CHUNK, _CHUNK)], sem))
    for c in range(4):
        for g in range(_RPW // _L):
            idx_v[pl.ds(c * _RPW + g * _L, _L)] = (
                ord_v[pl.ds(g * _L, _L)] + c * _N)
        for g in range(_RPW // _CHUNK):
            copies.append(pltpu.async_copy(
                btfin_hbm.at[idx_v.at[pl.ds(c * _RPW + g * _CHUNK, _CHUNK)]],
                btl_v.at[pl.ds(c * _RPW + g * _CHUNK, _CHUNK)], sem))
    for cp in copies:
        cp.wait()
    outs = [pltpu.async_copy(btl_v.at[pl.ds(c * _RPW, _RPW)],
                             btf_hbm.at[pl.ds(c * _N_PAD + base, _RPW)], sem)
            for c in range(4)]
    outs.append(pltpu.async_copy(spl_v, sp_hbm.at[pl.ds(base, _RPW)], sem))
    for cp in outs:
        cp.wait()


_gather_layout = functools.partial(
    pl.kernel,
    mesh=plsc.VectorSubcoreMesh(core_axis_name="c", subcore_axis_name="s"),
    out_type=[
        jax.ShapeDtypeStruct((4 * _N_PAD,), jnp.float32),
        jax.ShapeDtypeStruct((_N_PAD,), jnp.float32),
    ],
    scratch_types=[
        pltpu.VMEM((_RPW,), jnp.int32),
        pltpu.VMEM((4 * _RPW,), jnp.int32),
        pltpu.VMEM((4 * _RPW,), jnp.float32),
        pltpu.VMEM((_RPW,), jnp.float32),
        pltpu.SemaphoreType.DMA,
    ],
)(_gather_body)


def _iou_tile_mask(cy1, cx1, cy2, cx2, carea, ry1, rx1, ry2, rx2, rarea):
    tly = jnp.maximum(cy1, ry1)
    tlx = jnp.maximum(cx1, rx1)
    bry = jnp.minimum(cy2, ry2)
    brx = jnp.minimum(cx2, rx2)
    h = jnp.maximum(bry - tly, 0.0)
    w = jnp.maximum(brx - tlx, 0.0)
    inter = h * w
    iou = inter / (carea + rarea - inter + 1e-9)
    return (iou > _IOU_T).astype(jnp.float32)


def _nms_body(bt_ref, s_ref, out_ref, keep_ref):
    keep_ref[...] = (s_ref[...] > _SCORE_T).astype(jnp.float32)

    row_id = jax.lax.broadcasted_iota(jnp.int32, (_BLK, _BLK), 0)
    col_id = jax.lax.broadcasted_iota(jnp.int32, (_BLK, _BLK), 1)
    tri = (col_id < row_id).astype(jnp.float32)
    eyeb = (row_id == col_id).astype(jnp.float32)

    def coord_rows(jb):
        rb = bt_ref[:, pl.ds(jb, _BLK)]
        return rb[0:1, :], rb[1:2, :], rb[2:3, :], rb[3:4, :]

    def t_col(rowvec):
        return jax.lax.dot_general(eyeb, rowvec, (((1,), (1,)), ((), ())),
                                   precision=jax.lax.Precision.HIGHEST,
                                   preferred_element_type=jnp.float32)

    for k in range(_NBLK):
        base = k * _BLK
        by1, bx1, by2, bx2 = coord_rows(base)
        cy1, cx1, cy2, cx2 = t_col(by1), t_col(bx1), t_col(by2), t_col(bx2)
        carea = (cy2 - cy1) * (cx2 - cx1)
        k0 = keep_ref[pl.ds(base, _BLK), :]

        def cross_body(j, acc, cy1=cy1, cx1=cx1, cy2=cy2, cx2=cx2,
                       carea=carea):
            ry1, rx1, ry2, rx2 = coord_rows(j * _BLK)
            rarea = (ry2 - ry1) * (rx2 - rx1)
            m = _iou_tile_mask(cy1, cx1, cy2, cx2, carea,
                               ry1, rx1, ry2, rx2, rarea)
            kprev = keep_ref[pl.ds(j * _BLK, _BLK), :]
            return acc + jax.lax.dot(m, kprev,
                                     preferred_element_type=jnp.float32)

        sup = jax.lax.fori_loop(0, k, cross_body,
                                jnp.zeros((_BLK, 1), jnp.float32))
        k0_eff = jnp.where(sup > 0.5, 0.0, k0)

        rarea = (by2 - by1) * (bx2 - bx1)
        m_self = _iou_tile_mask(cy1, cx1, cy2, cx2, carea,
                                by1, bx1, by2, bx2, rarea) * tri

        def fp_cond(st):
            t, _, changed = st
            return changed & (t < _BLK + 2)

        def fp_body(st, m_self=m_self, k0_eff=k0_eff):
            t, kc, _ = st
            sup2 = jax.lax.dot(m_self, kc, preferred_element_type=jnp.float32)
            kn = jnp.where(sup2 > 0.5, 0.0, k0_eff)
            return (t + 1, kn, jnp.any(kn != kc))

        _, kfin, _ = jax.lax.while_loop(
            fp_cond, fp_body, (0, k0_eff, jnp.bool_(True)))
        keep_ref[pl.ds(base, _BLK), :] = kfin

        bblk = bt_ref[:, pl.ds(base, _BLK)]
        brows = jax.lax.dot_general(eyeb, bblk, (((1,), (1,)), ((), ())),
                                    precision=jax.lax.Precision.HIGHEST,
                                    preferred_element_type=jnp.float32)
        srows = s_ref[pl.ds(base, _BLK), :]
        out_ref[pl.ds(base, _BLK), :] = (
            jnp.concatenate([brows, srows], axis=1) * kfin)


def _run_nms(btp, sp, interpret=False):
    return pl.pallas_call(
        _nms_body,
        out_shape=jax.ShapeDtypeStruct((_N_PAD, 5), jnp.float32),
        scratch_shapes=[pltpu.VMEM((_N_PAD, 1), jnp.float32)],
        interpret=interpret,
    )(btp, sp)


def kernel(boxes, scores):
    order = jnp.argsort(-scores)
    orderp = jnp.pad(order, ((0, _N_PAD - _N),))
    btfin = boxes.T.reshape(4 * _N)
    btf, spf = _gather_layout(btfin, scores, orderp)
    out = _run_nms(btf.reshape(4, _N_PAD), spf[:, None])
    return out[:_N]


# speedup vs baseline: 1.1283x; 1.0172x over previous
import functools

import jax
import jax.numpy as jnp
from jax import lax
from jax.experimental import pallas as pl
from jax.experimental.pallas import tpu as pltpu
from jax.experimental.pallas import tpu_sc as plsc

_N = 5000
_N_PAD = 5120
_BLK = 1024
_NBLK = _N_PAD // _BLK
_IOU_T = 0.3
_SCORE_T = 0.05

_NC = 2
_NS = 16
_L = 16
_NW = _NC * _NS
_RPW = _N_PAD // _NW
_CHUNK = 80


def _gather_body(btfin_hbm, scores_hbm, order_hbm, btf_hbm, sp_hbm,
                 ord_v, idx_v, btl_v, spl_v, sem):
    wid = lax.axis_index("s") * _NC + lax.axis_index("c")
    base = wid * _RPW
    pltpu.sync_copy(order_hbm.at[pl.ds(base, _RPW)], ord_v)
    copies = []
    for g in range(_RPW // _CHUNK):
        copies.append(pltpu.async_copy(
            scores_hbm.at[ord_v.at[pl.ds(g * _CHUNK, _CHUNK)]],
            spl_v.at[pl.ds(g * _CHUNK, _CHUNK)], sem))
    for c in range(4):
        for g in range(_RPW // _L):
            idx_v[pl.ds(c * _RPW + g * _L, _L)] = (
                ord_v[pl.ds(g * _L, _L)] + c * _N)
        for g in range(_RPW // _CHUNK):
            copies.append(pltpu.async_copy(
                btfin_hbm.at[idx_v.at[pl.ds(c * _RPW + g * _CHUNK, _CHUNK)]],
                btl_v.at[pl.ds(c * _RPW + g * _CHUNK, _CHUNK)], sem))
    for cp in copies:
        cp.wait()
    outs = [pltpu.async_copy(btl_v.at[pl.ds(c * _RPW, _RPW)],
                             btf_hbm.at[pl.ds(c * _N_PAD + base, _RPW)], sem)
            for c in range(4)]
    outs.append(pltpu.async_copy(spl_v, sp_hbm.at[pl.ds(base, _RPW)], sem))
    for cp in outs:
        cp.wait()


_gather_layout = functools.partial(
    pl.kernel,
    mesh=plsc.VectorSubcoreMesh(core_axis_name="c", subcore_axis_name="s"),
    out_type=[
        jax.ShapeDtypeStruct((4 * _N_PAD,), jnp.float32),
        jax.ShapeDtypeStruct((_N_PAD,), jnp.float32),
    ],
    scratch_types=[
        pltpu.VMEM((_RPW,), jnp.int32),
        pltpu.VMEM((4 * _RPW,), jnp.int32),
        pltpu.VMEM((4 * _RPW,), jnp.float32),
        pltpu.VMEM((_RPW,), jnp.float32),
        pltpu.SemaphoreType.DMA,
    ],
)(_gather_body)


def _iou_tile_mask(cy1, cx1, cy2, cx2, carea, ry1, rx1, ry2, rx2, rarea):
    tly = jnp.maximum(cy1, ry1)
    tlx = jnp.maximum(cx1, rx1)
    bry = jnp.minimum(cy2, ry2)
    brx = jnp.minimum(cx2, rx2)
    h = jnp.maximum(bry - tly, 0.0)
    w = jnp.maximum(brx - tlx, 0.0)
    inter = h * w
    iou = inter / (carea + rarea - inter + 1e-9)
    return (iou > _IOU_T).astype(jnp.float32)


def _nms_body(bt_ref, s_ref, out_ref, keep_ref):
    keep_ref[...] = (s_ref[...] > _SCORE_T).astype(jnp.float32)

    row_id = jax.lax.broadcasted_iota(jnp.int32, (_BLK, _BLK), 0)
    col_id = jax.lax.broadcasted_iota(jnp.int32, (_BLK, _BLK), 1)
    tri = (col_id < row_id).astype(jnp.float32)
    eyeb = (row_id == col_id).astype(jnp.float32)

    def coord_rows(jb):
        rb = bt_ref[:, pl.ds(jb, _BLK)]
        return rb[0:1, :], rb[1:2, :], rb[2:3, :], rb[3:4, :]

    def t_col(rowvec):
        return jax.lax.dot_general(eyeb, rowvec, (((1,), (1,)), ((), ())),
                                   precision=jax.lax.Precision.HIGHEST,
                                   preferred_element_type=jnp.float32)

    for k in range(_NBLK):
        base = k * _BLK
        by1, bx1, by2, bx2 = coord_rows(base)
        cy1, cx1, cy2, cx2 = t_col(by1), t_col(bx1), t_col(by2), t_col(bx2)
        carea = (cy2 - cy1) * (cx2 - cx1)
        k0 = keep_ref[pl.ds(base, _BLK), :]

        def cross_body(j, acc, cy1=cy1, cx1=cx1, cy2=cy2, cx2=cx2,
                       carea=carea):
            ry1, rx1, ry2, rx2 = coord_rows(j * _BLK)
            rarea = (ry2 - ry1) * (rx2 - rx1)
            m = _iou_tile_mask(cy1, cx1, cy2, cx2, carea,
                               ry1, rx1, ry2, rx2, rarea)
            kprev = keep_ref[pl.ds(j * _BLK, _BLK), :]
            return acc + jax.lax.dot(m, kprev,
                                     preferred_element_type=jnp.float32)

        sup = jax.lax.fori_loop(0, k, cross_body,
                                jnp.zeros((_BLK, 1), jnp.float32))
        k0_eff = jnp.where(sup > 0.5, 0.0, k0)

        rarea = (by2 - by1) * (bx2 - bx1)
        m_self = _iou_tile_mask(cy1, cx1, cy2, cx2, carea,
                                by1, bx1, by2, bx2, rarea) * tri

        def fp_cond(st):
            t, _, changed = st
            return changed & (t < _BLK + 2)

        def fp_body(st, m_self=m_self, k0_eff=k0_eff):
            t, kc, _ = st
            sup2 = jax.lax.dot(m_self, kc, preferred_element_type=jnp.float32)
            kn = jnp.where(sup2 > 0.5, 0.0, k0_eff)
            return (t + 1, kn, jnp.any(kn != kc))

        _, kfin, _ = jax.lax.while_loop(
            fp_cond, fp_body, (0, k0_eff, jnp.bool_(True)))
        keep_ref[pl.ds(base, _BLK), :] = kfin

        bblk = bt_ref[:, pl.ds(base, _BLK)]
        brows = jax.lax.dot_general(eyeb, bblk, (((1,), (1,)), ((), ())),
                                    precision=jax.lax.Precision.HIGHEST,
                                    preferred_element_type=jnp.float32)
        srows = s_ref[pl.ds(base, _BLK), :]
        out_ref[pl.ds(base, _BLK), :] = (
            jnp.concatenate([brows, srows], axis=1) * kfin)


def _run_nms(btp, sp):
    return pl.pallas_call(
        _nms_body,
        out_shape=jax.ShapeDtypeStruct((_N_PAD, 5), jnp.float32),
        scratch_shapes=[pltpu.VMEM((_N_PAD, 1), jnp.float32)],
    )(btp, sp)


def kernel(boxes, scores):
    order = jnp.argsort(-scores)
    orderp = jnp.pad(order, ((0, _N_PAD - _N),))
    btfin = boxes.T.reshape(4 * _N)
    btf, spf = _gather_layout(btfin, scores, orderp)
    out = _run_nms(btf.reshape(4, _N_PAD), spf[:, None])
    return out[:_N]
